# baseline (speedup 1.0000x reference)
import functools

import jax
import jax.numpy as jnp
from jax import lax
from jax.experimental import pallas as pl
from jax.experimental.pallas import tpu as pltpu
from jax.experimental.pallas import tpu_sc as plsc

N = 10000
E = 640000
D_IN = 128
H = 64
G = 64

NPAD = 10240
BLK = 512
GRID = NPAD // BLK

NT = 32
CH = 128
NB = 4
IB = 8
NCHUNK = 160
NBLK = NCHUNK // IB
NALLOC = NCHUNK + IB
RPT = NPAD // 16
CBP = 6
DEGW = 16

_mesh = plsc.VectorSubcoreMesh(core_axis_name="c", subcore_axis_name="s")


def _sc_deg(dst3d):
    @functools.partial(
        pl.kernel,
        mesh=_mesh,
        out_type=jax.ShapeDtypeStruct((2, NPAD, DEGW), jnp.float32),
        compiler_params=pltpu.CompilerParams(use_tc_tiling_on_sc=False),
        scratch_types=[
            pltpu.VMEM((NALLOC, CH), jnp.int32),
            pltpu.VMEM((CH, DEGW), jnp.float32),
            pltpu.VMEM((64, DEGW), jnp.float32),
            pltpu.VMEM_SHARED((NPAD, DEGW), jnp.float32),
            pltpu.SemaphoreType.DMA,
            pltpu.SemaphoreType.DMA,
        ],
    )
    def k(dst_ref, out_ref, dst_all, ones, zb, acc, isem, ssem):
        c = lax.axis_index("c")
        s = lax.axis_index("s")
        blk = c * 16 + s

        idx_cp = pltpu.async_copy(dst_ref.at[blk], dst_all, isem)

        @pl.loop(0, 64)
        def _(r):
            zb.at[r][pl.ds(0, 16)] = jnp.zeros((16,), jnp.float32)

        @pl.loop(0, CH)
        def _(r):
            ones.at[r][pl.ds(0, 16)] = jnp.ones((16,), jnp.float32)

        @pl.loop(0, RPT, step=64)
        def _(r0):
            pltpu.sync_copy(zb, acc.at[pl.ds(s * RPT + r0, 64)])

        idx_cp.wait()
        plsc.subcore_barrier()

        @pl.loop(0, NALLOC // 8)
        def _(g):
            base = g * 8
            for kk in range(8):
                pltpu.async_copy(ones, acc.at[dst_all.at[base + kk]], ssem,
                                 add=True)
            for kk in range(8):
                pltpu.make_async_copy(ones, acc.at[dst_all.at[base + kk]],
                                      ssem).wait()

        plsc.subcore_barrier()
        pltpu.sync_copy(acc.at[pl.ds(s * RPT, RPT)],
                        out_ref.at[c, pl.ds(s * RPT, RPT)])

    return k(dst3d)


def _sc_prop(t_hbm, src3d, dst3d):
    @functools.partial(
        pl.kernel,
        mesh=_mesh,
        out_type=jax.ShapeDtypeStruct((2, NPAD, H), jnp.float32),
        compiler_params=pltpu.CompilerParams(use_tc_tiling_on_sc=False),
        scratch_types=(
            [pltpu.VMEM((IB, CH), jnp.int32)] * 4
            + [pltpu.VMEM((CH, H), jnp.float32)] * NB
            + [pltpu.VMEM((64, H), jnp.float32)]
            + [pltpu.VMEM_SHARED((NPAD, H), jnp.float32)]
            + [pltpu.VMEM_SHARED((NPAD, H), jnp.float32)]
            + [pltpu.SemaphoreType.DMA] * (2 * NB + 2)
        ),
    )
    def k(t_ref, src_ref, dst_ref, out_ref, *sc):
        sidx = sc[0:2]
        didx = sc[2:4]
        rows = sc[4:4 + NB]
        zb = sc[4 + NB]
        acc = sc[5 + NB]
        t_sp = sc[6 + NB]
        gsems = sc[7 + NB:7 + 2 * NB]
        ssems = sc[7 + 2 * NB:7 + 3 * NB]
        isem = sc[7 + 3 * NB]
        stsem = sc[8 + 3 * NB]

        c = lax.axis_index("c")
        s = lax.axis_index("s")
        blk = c * 16 + s

        stage_cp = pltpu.async_copy(t_ref.at[pl.ds(s * RPT, RPT)],
                                    t_sp.at[pl.ds(s * RPT, RPT)], stsem)
        s0_cp = pltpu.async_copy(src_ref.at[blk, pl.ds(0, IB)], sidx[0], isem)
        d0_cp = pltpu.async_copy(dst_ref.at[blk, pl.ds(0, IB)], didx[0], isem)

        @pl.loop(0, 64)
        def _(r):
            @pl.loop(0, H, step=16)
            def _(k2):
                zb.at[r][pl.ds(k2, 16)] = jnp.zeros((16,), jnp.float32)

        for r0 in range(0, RPT, 64):
            pltpu.async_copy(zb, acc.at[pl.ds(s * RPT + r0, 64)], ssems[0])
        for r0 in range(0, RPT, 64):
            pltpu.make_async_copy(zb, acc.at[pl.ds(s * RPT + r0, 64)],
                                  ssems[0]).wait()

        stage_cp.wait()
        s0_cp.wait()
        d0_cp.wait()
        plsc.subcore_barrier()

        for b in range(NB):
            pltpu.async_copy(t_sp.at[sidx[0].at[b]], rows[b], gsems[b])

        def nb_group(gref, cur_s, cur_d, nxt_s, nxt_d, j0, j0n):
            for b in range(NB):
                pltpu.make_async_copy(gref.at[cur_s.at[j0 + b]], rows[b],
                                      gsems[b]).wait()
                pltpu.async_copy(rows[b], acc.at[cur_d.at[j0 + b]], ssems[b],
                                 add=True)
            for b in range(NB):
                pltpu.make_async_copy(rows[b], acc.at[cur_d.at[j0 + b]],
                                      ssems[b]).wait()
                pltpu.async_copy(gref.at[nxt_s.at[j0n + b]], rows[b],
                                 gsems[b])

        def block_pair(kk, gref):
            for par in range(2):
                k2 = kk * 2 + par
                cur_s, cur_d = sidx[par], didx[par]
                oth_s, oth_d = sidx[1 - par], didx[1 - par]
                base = k2 * IB
                pltpu.async_copy(src_ref.at[blk, pl.ds(base + IB, IB)],
                                 oth_s, isem)
                pltpu.async_copy(dst_ref.at[blk, pl.ds(base + IB, IB)],
                                 oth_d, isem)
                nb_group(gref, cur_s, cur_d, cur_s, cur_d, 0, NB)
                pltpu.make_async_copy(src_ref.at[blk, pl.ds(base + IB, IB)],
                                      oth_s, isem).wait()
                pltpu.make_async_copy(dst_ref.at[blk, pl.ds(base + IB, IB)],
                                      oth_d, isem).wait()
                nb_group(gref, cur_s, cur_d, oth_s, oth_d, NB, 0)

        @pl.loop(0, CBP)
        def _(kk):
            block_pair(kk, t_sp)

        @pl.loop(CBP, NBLK // 2)
        def _(kk):
            block_pair(kk, t_ref)

        for b in range(NB):
            pltpu.make_async_copy(t_sp.at[sidx[0].at[b]], rows[b],
                                  gsems[b]).wait()

        plsc.subcore_barrier()
        pltpu.sync_copy(acc.at[pl.ds(s * RPT, RPT)],
                        out_ref.at[c, pl.ds(s * RPT, RPT)])

    return k(t_hbm, src3d, dst3d)


def _tc_u1(x_p, W1):
    def body(x_ref, w_ref, u_ref):
        u_ref[...] = jnp.dot(x_ref[...], w_ref[...],
                             preferred_element_type=jnp.float32)

    return pl.pallas_call(
        body,
        grid=(GRID,),
        in_specs=[
            pl.BlockSpec((BLK, D_IN), lambda i: (i, 0)),
            pl.BlockSpec((D_IN, H), lambda i: (0, 0)),
        ],
        out_specs=pl.BlockSpec((BLK, H), lambda i: (i, 0)),
        out_shape=jax.ShapeDtypeStruct((NPAD, H), jnp.float32),
    )(x_p, W1)


def _tc_dinv_t1(degacc, u1):
    def body(deg_ref, u_ref, dinv_ref, t_ref):
        i = pl.program_id(0)
        d = deg_ref[0, :, 0:1] + deg_ref[1, :, 0:1] + 1.0
        rows = lax.broadcasted_iota(jnp.int32, (BLK, 1), 0) + i * BLK
        dinv = jnp.where(rows < N, lax.rsqrt(d), 0.0)
        dinv_ref[...] = dinv
        t_ref[...] = u_ref[...] * dinv

    return pl.pallas_call(
        body,
        grid=(GRID,),
        in_specs=[
            pl.BlockSpec((2, BLK, DEGW), lambda i: (0, i, 0)),
            pl.BlockSpec((BLK, H), lambda i: (i, 0)),
        ],
        out_specs=[
            pl.BlockSpec((BLK, 1), lambda i: (i, 0)),
            pl.BlockSpec((BLK, H), lambda i: (i, 0)),
        ],
        out_shape=[
            jax.ShapeDtypeStruct((NPAD, 1), jnp.float32),
            jax.ShapeDtypeStruct((NPAD, H), jnp.float32),
        ],
    )(degacc, u1)


def _tc_mid(acc, t, dinv, b, Wn):
    def body(a_ref, t_ref, dinv_ref, b_ref, w_ref, o_ref):
        z = (a_ref[0] + a_ref[1] + t_ref[...]) * dinv_ref[...] + b_ref[...]
        z = jnp.maximum(z, 0.0)
        u = jnp.dot(z, w_ref[...], preferred_element_type=jnp.float32)
        o_ref[...] = u * dinv_ref[...]

    return pl.pallas_call(
        body,
        grid=(GRID,),
        in_specs=[
            pl.BlockSpec((2, BLK, H), lambda i: (0, i, 0)),
            pl.BlockSpec((BLK, H), lambda i: (i, 0)),
            pl.BlockSpec((BLK, 1), lambda i: (i, 0)),
            pl.BlockSpec((1, H), lambda i: (0, 0)),
            pl.BlockSpec((H, H), lambda i: (0, 0)),
        ],
        out_specs=pl.BlockSpec((BLK, H), lambda i: (i, 0)),
        out_shape=jax.ShapeDtypeStruct((NPAD, H), jnp.float32),
    )(acc, t, dinv, b, Wn)


def _tc_final(acc, t, dinv, b3, batch2d, Wo, bo):
    def body(a_ref, t_ref, dinv_ref, b_ref, batch_ref, wo_ref, bo_ref,
             o_ref, pooled):
        i = pl.program_id(0)
        z = (a_ref[0] + a_ref[1] + t_ref[...]) * dinv_ref[...] + b_ref[...]
        z = jnp.maximum(z, 0.0)
        rows = lax.broadcasted_iota(jnp.int32, (BLK, 1), 0) + i * BLK
        z = jnp.where(rows < N, z, 0.0)
        segs = lax.broadcasted_iota(jnp.int32, (G, 1), 0)
        oh = (segs == batch_ref[...]).astype(jnp.float32)
        contrib = jnp.dot(oh, z, preferred_element_type=jnp.float32)

        @pl.when(i == 0)
        def _():
            pooled[...] = contrib

        @pl.when(i > 0)
        def _():
            pooled[...] = pooled[...] + contrib

        @pl.when(i == GRID - 1)
        def _():
            o_ref[...] = jnp.dot(pooled[...], wo_ref[...],
                                 preferred_element_type=jnp.float32) + bo_ref[...]

    return pl.pallas_call(
        body,
        grid=(GRID,),
        in_specs=[
            pl.BlockSpec((2, BLK, H), lambda i: (0, i, 0)),
            pl.BlockSpec((BLK, H), lambda i: (i, 0)),
            pl.BlockSpec((BLK, 1), lambda i: (i, 0)),
            pl.BlockSpec((1, H), lambda i: (0, 0)),
            pl.BlockSpec((1, BLK), lambda i: (0, i)),
            pl.BlockSpec((H, 1), lambda i: (0, 0)),
            pl.BlockSpec((1, 1), lambda i: (0, 0)),
        ],
        out_specs=pl.BlockSpec((G, 1), lambda i: (0, 0)),
        out_shape=jax.ShapeDtypeStruct((G, 1), jnp.float32),
        scratch_shapes=[pltpu.VMEM((G, H), jnp.float32)],
    )(acc, t, dinv, b3, batch2d, Wo, bo)


def kernel(x, edge_index, batch, W1, b1, W2, b2, W3, b3, Wo, bo):
    x_p = jnp.zeros((NPAD, D_IN), jnp.float32).at[:N].set(x)

    src = edge_index[0]
    dst = edge_index[1]
    def _part(e):
        e2 = jnp.full((NT * NCHUNK * CH,), N, jnp.int32).at[:E].set(e)
        e2 = e2.reshape(NT, NCHUNK * CH)
        pad = jnp.full((NT, (NALLOC - NCHUNK) * CH), N, jnp.int32)
        return jnp.concatenate([e2, pad], axis=1).reshape(NT, NALLOC, CH)

    src3d = _part(src)
    dst3d = _part(dst)

    batch2d = jnp.zeros((1, NPAD), jnp.int32).at[0, :N].set(batch)

    b1r = b1.reshape(1, H)
    b2r = b2.reshape(1, H)
    b3r = b3.reshape(1, H)
    bor = bo.reshape(1, 1)

    u1 = _tc_u1(x_p, W1)
    degacc = _sc_deg(dst3d)
    dinv, t1 = _tc_dinv_t1(degacc, u1)
    a1 = _sc_prop(t1, src3d, dst3d)
    t2 = _tc_mid(a1, t1, dinv, b1r, W2)
    a2 = _sc_prop(t2, src3d, dst3d)
    t3 = _tc_mid(a2, t2, dinv, b2r, W3)
    a3 = _sc_prop(t3, src3d, dst3d)
    return _tc_final(a3, t3, dinv, b3r, batch2d, Wo, bor)


# speedup vs baseline: 2.6485x; 2.6485x over previous
import functools

import jax
import jax.numpy as jnp
from jax import lax
from jax.experimental import pallas as pl
from jax.experimental.pallas import tpu as pltpu
from jax.experimental.pallas import tpu_sc as plsc

N = 10000
E = 640000
D_IN = 128
H = 64
G = 64

NPAD = 10240
BLK = 512
GRID = NPAD // BLK

NT = 32
CH = 128
NB = 4
IB = 16
NCHUNK = 160
NBLK = NCHUNK // IB
NALLOC = NCHUNK + IB
RPT = NPAD // 16
CBP = NBLK // 2
DEGW = 16

_mesh = plsc.VectorSubcoreMesh(core_axis_name="c", subcore_axis_name="s")


def _sc_deg(dst3d):
    @functools.partial(
        pl.kernel,
        mesh=_mesh,
        out_type=jax.ShapeDtypeStruct((2, NPAD, DEGW), jnp.float32),
        compiler_params=pltpu.CompilerParams(use_tc_tiling_on_sc=False),
        scratch_types=[
            pltpu.VMEM((NALLOC, CH), jnp.int32),
            pltpu.VMEM((CH, DEGW), jnp.float32),
            pltpu.VMEM((64, DEGW), jnp.float32),
            pltpu.VMEM_SHARED((NPAD, DEGW), jnp.float32),
            pltpu.SemaphoreType.DMA,
            pltpu.SemaphoreType.DMA,
        ],
    )
    def k(dst_ref, out_ref, dst_all, ones, zb, acc, isem, ssem):
        c = lax.axis_index("c")
        s = lax.axis_index("s")
        blk = c * 16 + s

        idx_cp = pltpu.async_copy(dst_ref.at[blk], dst_all, isem)

        @pl.loop(0, 64)
        def _(r):
            zb.at[r][pl.ds(0, 16)] = jnp.zeros((16,), jnp.float32)

        @pl.loop(0, CH)
        def _(r):
            ones.at[r][pl.ds(0, 16)] = jnp.ones((16,), jnp.float32)

        @pl.loop(0, RPT, step=64)
        def _(r0):
            pltpu.sync_copy(zb, acc.at[pl.ds(s * RPT + r0, 64)])

        idx_cp.wait()
        plsc.subcore_barrier()

        @pl.loop(0, NALLOC // 8)
        def _(g):
            base = g * 8
            for kk in range(8):
                pltpu.async_copy(ones, acc.at[dst_all.at[base + kk]], ssem,
                                 add=True)
            for kk in range(8):
                pltpu.make_async_copy(ones, acc.at[dst_all.at[base + kk]],
                                      ssem).wait()

        plsc.subcore_barrier()
        pltpu.sync_copy(acc.at[pl.ds(s * RPT, RPT)],
                        out_ref.at[c, pl.ds(s * RPT, RPT)])

    return k(dst3d)


def _sc_prop(t_hbm, src3d, dst3d):
    @functools.partial(
        pl.kernel,
        mesh=_mesh,
        out_type=jax.ShapeDtypeStruct((2, NPAD, H), jnp.float32),
        compiler_params=pltpu.CompilerParams(use_tc_tiling_on_sc=False),
        scratch_types=(
            [pltpu.VMEM((IB, CH), jnp.int32)] * 4
            + [pltpu.VMEM((CH, H), jnp.float32)] * NB
            + [pltpu.VMEM((32, H), jnp.float32)]
            + [pltpu.VMEM_SHARED((NPAD, H), jnp.float32)]
            + [pltpu.VMEM_SHARED((NPAD, H), jnp.float32)]
            + [pltpu.SemaphoreType.DMA] * (2 * NB + 2)
        ),
    )
    def k(t_ref, src_ref, dst_ref, out_ref, *sc):
        sidx = sc[0:2]
        didx = sc[2:4]
        rows = sc[4:4 + NB]
        zb = sc[4 + NB]
        acc = sc[5 + NB]
        t_sp = sc[6 + NB]
        gsems = sc[7 + NB:7 + 2 * NB]
        ssems = sc[7 + 2 * NB:7 + 3 * NB]
        isem = sc[7 + 3 * NB]
        stsem = sc[8 + 3 * NB]

        c = lax.axis_index("c")
        s = lax.axis_index("s")
        blk = c * 16 + s

        stage_cp = pltpu.async_copy(t_ref.at[pl.ds(s * RPT, RPT)],
                                    t_sp.at[pl.ds(s * RPT, RPT)], stsem)
        s0_cp = pltpu.async_copy(src_ref.at[blk, pl.ds(0, IB)], sidx[0], isem)
        d0_cp = pltpu.async_copy(dst_ref.at[blk, pl.ds(0, IB)], didx[0], isem)

        @pl.loop(0, 32)
        def _(r):
            @pl.loop(0, H, step=16)
            def _(k2):
                zb.at[r][pl.ds(k2, 16)] = jnp.zeros((16,), jnp.float32)

        for r0 in range(0, RPT, 32):
            pltpu.async_copy(zb, acc.at[pl.ds(s * RPT + r0, 32)], ssems[0])
        for r0 in range(0, RPT, 32):
            pltpu.make_async_copy(zb, acc.at[pl.ds(s * RPT + r0, 32)],
                                  ssems[0]).wait()

        stage_cp.wait()
        s0_cp.wait()
        d0_cp.wait()
        plsc.subcore_barrier()

        for b in range(NB):
            pltpu.async_copy(t_sp.at[sidx[0].at[b]], rows[b], gsems[b])

        def nb_group(gref, cur_s, cur_d, nxt_s, nxt_d, j0, j0n):
            for b in range(NB):
                pltpu.make_async_copy(gref.at[cur_s.at[j0 + b]], rows[b],
                                      gsems[b]).wait()
                pltpu.async_copy(rows[b], acc.at[cur_d.at[j0 + b]], ssems[b],
                                 add=True)
            for b in range(NB):
                pltpu.make_async_copy(rows[b], acc.at[cur_d.at[j0 + b]],
                                      ssems[b]).wait()
                pltpu.async_copy(gref.at[nxt_s.at[j0n + b]], rows[b],
                                 gsems[b])

        def block_pair(kk, gref):
            for par in range(2):
                k2 = kk * 2 + par
                cur_s, cur_d = sidx[par], didx[par]
                oth_s, oth_d = sidx[1 - par], didx[1 - par]
                base = k2 * IB
                pltpu.async_copy(src_ref.at[blk, pl.ds(base + IB, IB)],
                                 oth_s, isem)
                pltpu.async_copy(dst_ref.at[blk, pl.ds(base + IB, IB)],
                                 oth_d, isem)
                for j0 in range(0, IB - NB, NB):
                    nb_group(gref, cur_s, cur_d, cur_s, cur_d, j0, j0 + NB)
                pltpu.make_async_copy(src_ref.at[blk, pl.ds(base + IB, IB)],
                                      oth_s, isem).wait()
                pltpu.make_async_copy(dst_ref.at[blk, pl.ds(base + IB, IB)],
                                      oth_d, isem).wait()
                nb_group(gref, cur_s, cur_d, oth_s, oth_d, IB - NB, 0)

        @pl.loop(0, CBP)
        def _(kk):
            block_pair(kk, t_sp)

        @pl.loop(CBP, NBLK // 2)
        def _(kk):
            block_pair(kk, t_ref)

        for b in range(NB):
            pltpu.make_async_copy(t_sp.at[sidx[0].at[b]], rows[b],
                                  gsems[b]).wait()

        plsc.subcore_barrier()
        pltpu.sync_copy(acc.at[pl.ds(s * RPT, RPT)],
                        out_ref.at[c, pl.ds(s * RPT, RPT)])

    return k(t_hbm, src3d, dst3d)


def _tc_u1(x_p, W1):
    def body(x_ref, w_ref, u_ref):
        u_ref[...] = jnp.dot(x_ref[...], w_ref[...],
                             preferred_element_type=jnp.float32)

    return pl.pallas_call(
        body,
        grid=(GRID,),
        in_specs=[
            pl.BlockSpec((BLK, D_IN), lambda i: (i, 0)),
            pl.BlockSpec((D_IN, H), lambda i: (0, 0)),
        ],
        out_specs=pl.BlockSpec((BLK, H), lambda i: (i, 0)),
        out_shape=jax.ShapeDtypeStruct((NPAD, H), jnp.float32),
    )(x_p, W1)


def _tc_dinv_t1(degacc, u1):
    def body(deg_ref, u_ref, dinv_ref, t_ref):
        i = pl.program_id(0)
        d = deg_ref[0, :, 0:1] + deg_ref[1, :, 0:1] + 1.0
        rows = lax.broadcasted_iota(jnp.int32, (BLK, 1), 0) + i * BLK
        dinv = jnp.where(rows < N, lax.rsqrt(d), 0.0)
        dinv_ref[...] = dinv
        t_ref[...] = u_ref[...] * dinv

    return pl.pallas_call(
        body,
        grid=(GRID,),
        in_specs=[
            pl.BlockSpec((2, BLK, DEGW), lambda i: (0, i, 0)),
            pl.BlockSpec((BLK, H), lambda i: (i, 0)),
        ],
        out_specs=[
            pl.BlockSpec((BLK, 1), lambda i: (i, 0)),
            pl.BlockSpec((BLK, H), lambda i: (i, 0)),
        ],
        out_shape=[
            jax.ShapeDtypeStruct((NPAD, 1), jnp.float32),
            jax.ShapeDtypeStruct((NPAD, H), jnp.float32),
        ],
    )(degacc, u1)


def _tc_mid(acc, t, dinv, b, Wn):
    def body(a_ref, t_ref, dinv_ref, b_ref, w_ref, o_ref):
        z = (a_ref[0] + a_ref[1] + t_ref[...]) * dinv_ref[...] + b_ref[...]
        z = jnp.maximum(z, 0.0)
        u = jnp.dot(z, w_ref[...], preferred_element_type=jnp.float32)
        o_ref[...] = u * dinv_ref[...]

    return pl.pallas_call(
        body,
        grid=(GRID,),
        in_specs=[
            pl.BlockSpec((2, BLK, H), lambda i: (0, i, 0)),
            pl.BlockSpec((BLK, H), lambda i: (i, 0)),
            pl.BlockSpec((BLK, 1), lambda i: (i, 0)),
            pl.BlockSpec((1, H), lambda i: (0, 0)),
            pl.BlockSpec((H, H), lambda i: (0, 0)),
        ],
        out_specs=pl.BlockSpec((BLK, H), lambda i: (i, 0)),
        out_shape=jax.ShapeDtypeStruct((NPAD, H), jnp.float32),
    )(acc, t, dinv, b, Wn)


def _tc_final(acc, t, dinv, b3, batch2d, Wo, bo):
    def body(a_ref, t_ref, dinv_ref, b_ref, batch_ref, wo_ref, bo_ref,
             o_ref, pooled):
        i = pl.program_id(0)
        z = (a_ref[0] + a_ref[1] + t_ref[...]) * dinv_ref[...] + b_ref[...]
        z = jnp.maximum(z, 0.0)
        rows = lax.broadcasted_iota(jnp.int32, (BLK, 1), 0) + i * BLK
        z = jnp.where(rows < N, z, 0.0)
        segs = lax.broadcasted_iota(jnp.int32, (G, 1), 0)
        oh = (segs == batch_ref[...]).astype(jnp.float32)
        contrib = jnp.dot(oh, z, preferred_element_type=jnp.float32)

        @pl.when(i == 0)
        def _():
            pooled[...] = contrib

        @pl.when(i > 0)
        def _():
            pooled[...] = pooled[...] + contrib

        @pl.when(i == GRID - 1)
        def _():
            o_ref[...] = jnp.dot(pooled[...], wo_ref[...],
                                 preferred_element_type=jnp.float32) + bo_ref[...]

    return pl.pallas_call(
        body,
        grid=(GRID,),
        in_specs=[
            pl.BlockSpec((2, BLK, H), lambda i: (0, i, 0)),
            pl.BlockSpec((BLK, H), lambda i: (i, 0)),
            pl.BlockSpec((BLK, 1), lambda i: (i, 0)),
            pl.BlockSpec((1, H), lambda i: (0, 0)),
            pl.BlockSpec((1, BLK), lambda i: (0, i)),
            pl.BlockSpec((H, 1), lambda i: (0, 0)),
            pl.BlockSpec((1, 1), lambda i: (0, 0)),
        ],
        out_specs=pl.BlockSpec((G, 1), lambda i: (0, 0)),
        out_shape=jax.ShapeDtypeStruct((G, 1), jnp.float32),
        scratch_shapes=[pltpu.VMEM((G, H), jnp.float32)],
    )(acc, t, dinv, b3, batch2d, Wo, bo)


def kernel(x, edge_index, batch, W1, b1, W2, b2, W3, b3, Wo, bo):
    x_p = jnp.zeros((NPAD, D_IN), jnp.float32).at[:N].set(x)

    src = edge_index[0]
    dst = edge_index[1]
    def _part(e):
        e2 = jnp.full((NT * NCHUNK * CH,), N, jnp.int32).at[:E].set(e)
        e2 = e2.reshape(NT, NCHUNK * CH)
        pad = jnp.full((NT, (NALLOC - NCHUNK) * CH), N, jnp.int32)
        return jnp.concatenate([e2, pad], axis=1).reshape(NT, NALLOC, CH)

    src3d = _part(src)
    dst3d = _part(dst)

    batch2d = jnp.zeros((1, NPAD), jnp.int32).at[0, :N].set(batch)

    b1r = b1.reshape(1, H)
    b2r = b2.reshape(1, H)
    b3r = b3.reshape(1, H)
    bor = bo.reshape(1, 1)

    u1 = _tc_u1(x_p, W1)
    degacc = _sc_deg(dst3d)
    dinv, t1 = _tc_dinv_t1(degacc, u1)
    a1 = _sc_prop(t1, src3d, dst3d)
    t2 = _tc_mid(a1, t1, dinv, b1r, W2)
    a2 = _sc_prop(t2, src3d, dst3d)
    t3 = _tc_mid(a2, t2, dinv, b2r, W3)
    a3 = _sc_prop(t3, src3d, dst3d)
    return _tc_final(a3, t3, dinv, b3r, batch2d, Wo, bor)


# speedup vs baseline: 2.7215x; 1.0276x over previous
import functools

import jax
import jax.numpy as jnp
from jax import lax
from jax.experimental import pallas as pl
from jax.experimental.pallas import tpu as pltpu
from jax.experimental.pallas import tpu_sc as plsc

N = 10000
E = 640000
D_IN = 128
H = 64
G = 64

NPAD = 10240
BLK = 512
GRID = NPAD // BLK

NT = 32
CH = 128
NB = 4
IB = 8
NCHUNK = 160
NBLK = NCHUNK // IB
NALLOC = NCHUNK + IB
RPT = NPAD // 16
CBP = NBLK // 2
DEGW = 16

_mesh = plsc.VectorSubcoreMesh(core_axis_name="c", subcore_axis_name="s")


def _sc_deg(dst3d):
    @functools.partial(
        pl.kernel,
        mesh=_mesh,
        out_type=jax.ShapeDtypeStruct((2, NPAD, DEGW), jnp.float32),
        compiler_params=pltpu.CompilerParams(use_tc_tiling_on_sc=False),
        scratch_types=[
            pltpu.VMEM((NALLOC, CH), jnp.int32),
            pltpu.VMEM((CH, DEGW), jnp.float32),
            pltpu.VMEM((64, DEGW), jnp.float32),
            pltpu.VMEM_SHARED((NPAD, DEGW), jnp.float32),
            pltpu.SemaphoreType.DMA,
            pltpu.SemaphoreType.DMA,
        ],
    )
    def k(dst_ref, out_ref, dst_all, ones, zb, acc, isem, ssem):
        c = lax.axis_index("c")
        s = lax.axis_index("s")
        blk = c * 16 + s

        idx_cp = pltpu.async_copy(dst_ref.at[blk], dst_all, isem)

        @pl.loop(0, 64)
        def _(r):
            zb.at[r][pl.ds(0, 16)] = jnp.zeros((16,), jnp.float32)

        @pl.loop(0, CH)
        def _(r):
            ones.at[r][pl.ds(0, 16)] = jnp.ones((16,), jnp.float32)

        @pl.loop(0, RPT, step=64)
        def _(r0):
            pltpu.sync_copy(zb, acc.at[pl.ds(s * RPT + r0, 64)])

        idx_cp.wait()
        plsc.subcore_barrier()

        @pl.loop(0, NALLOC // 8)
        def _(g):
            base = g * 8
            for kk in range(8):
                pltpu.async_copy(ones, acc.at[dst_all.at[base + kk]], ssem,
                                 add=True)
            for kk in range(8):
                pltpu.make_async_copy(ones, acc.at[dst_all.at[base + kk]],
                                      ssem).wait()

        plsc.subcore_barrier()
        pltpu.sync_copy(acc.at[pl.ds(s * RPT, RPT)],
                        out_ref.at[c, pl.ds(s * RPT, RPT)])

    return k(dst3d)


def _sc_prop(t_hbm, src3d, dst3d):
    @functools.partial(
        pl.kernel,
        mesh=_mesh,
        out_type=jax.ShapeDtypeStruct((2, NPAD, H), jnp.float32),
        compiler_params=pltpu.CompilerParams(use_tc_tiling_on_sc=False),
        scratch_types=(
            [pltpu.VMEM((IB, CH), jnp.int32)] * 4
            + [pltpu.VMEM((CH, H), jnp.float32)] * NB
            + [pltpu.VMEM((32, H), jnp.float32)]
            + [pltpu.VMEM_SHARED((NPAD, H), jnp.float32)]
            + [pltpu.VMEM_SHARED((NPAD, H), jnp.float32)]
            + [pltpu.SemaphoreType.DMA] * (2 * NB + 2)
        ),
    )
    def k(t_ref, src_ref, dst_ref, out_ref, *sc):
        sidx = sc[0:2]
        didx = sc[2:4]
        rows = sc[4:4 + NB]
        zb = sc[4 + NB]
        acc = sc[5 + NB]
        t_sp = sc[6 + NB]
        gsems = sc[7 + NB:7 + 2 * NB]
        ssems = sc[7 + 2 * NB:7 + 3 * NB]
        isem = sc[7 + 3 * NB]
        stsem = sc[8 + 3 * NB]

        c = lax.axis_index("c")
        s = lax.axis_index("s")
        blk = c * 16 + s

        stage_cp = pltpu.async_copy(t_ref.at[pl.ds(s * RPT, RPT)],
                                    t_sp.at[pl.ds(s * RPT, RPT)], stsem)
        s0_cp = pltpu.async_copy(src_ref.at[blk, pl.ds(0, IB)], sidx[0], isem)
        d0_cp = pltpu.async_copy(dst_ref.at[blk, pl.ds(0, IB)], didx[0], isem)

        @pl.loop(0, 32)
        def _(r):
            @pl.loop(0, H, step=16)
            def _(k2):
                zb.at[r][pl.ds(k2, 16)] = jnp.zeros((16,), jnp.float32)

        for r0 in range(0, RPT, 32):
            pltpu.async_copy(zb, acc.at[pl.ds(s * RPT + r0, 32)], ssems[0])
        for r0 in range(0, RPT, 32):
            pltpu.make_async_copy(zb, acc.at[pl.ds(s * RPT + r0, 32)],
                                  ssems[0]).wait()

        stage_cp.wait()
        s0_cp.wait()
        d0_cp.wait()
        plsc.subcore_barrier()

        for b in range(NB):
            pltpu.async_copy(t_sp.at[sidx[0].at[b]], rows[b], gsems[b])

        def nb_group(gref, cur_s, cur_d, nxt_s, nxt_d, j0, j0n):
            for b in range(NB):
                pltpu.make_async_copy(gref.at[cur_s.at[j0 + b]], rows[b],
                                      gsems[b]).wait()
                pltpu.async_copy(rows[b], acc.at[cur_d.at[j0 + b]], ssems[b],
                                 add=True)
            for b in range(NB):
                pltpu.make_async_copy(rows[b], acc.at[cur_d.at[j0 + b]],
                                      ssems[b]).wait()
                pltpu.async_copy(gref.at[nxt_s.at[j0n + b]], rows[b],
                                 gsems[b])

        def block_pair(kk, gref):
            for par in range(2):
                k2 = kk * 2 + par
                cur_s, cur_d = sidx[par], didx[par]
                oth_s, oth_d = sidx[1 - par], didx[1 - par]
                base = k2 * IB
                pltpu.async_copy(src_ref.at[blk, pl.ds(base + IB, IB)],
                                 oth_s, isem)
                pltpu.async_copy(dst_ref.at[blk, pl.ds(base + IB, IB)],
                                 oth_d, isem)
                for j0 in range(0, IB - NB, NB):
                    nb_group(gref, cur_s, cur_d, cur_s, cur_d, j0, j0 + NB)
                pltpu.make_async_copy(src_ref.at[blk, pl.ds(base + IB, IB)],
                                      oth_s, isem).wait()
                pltpu.make_async_copy(dst_ref.at[blk, pl.ds(base + IB, IB)],
                                      oth_d, isem).wait()
                nb_group(gref, cur_s, cur_d, oth_s, oth_d, IB - NB, 0)

        @pl.loop(0, CBP)
        def _(kk):
            block_pair(kk, t_sp)

        @pl.loop(CBP, NBLK // 2)
        def _(kk):
            block_pair(kk, t_ref)

        for b in range(NB):
            pltpu.make_async_copy(t_sp.at[sidx[0].at[b]], rows[b],
                                  gsems[b]).wait()

        plsc.subcore_barrier()
        pltpu.sync_copy(acc.at[pl.ds(s * RPT, RPT)],
                        out_ref.at[c, pl.ds(s * RPT, RPT)])

    return k(t_hbm, src3d, dst3d)


def _tc_u1(x_p, W1):
    def body(x_ref, w_ref, u_ref):
        u_ref[...] = jnp.dot(x_ref[...], w_ref[...],
                             preferred_element_type=jnp.float32)

    return pl.pallas_call(
        body,
        grid=(GRID,),
        in_specs=[
            pl.BlockSpec((BLK, D_IN), lambda i: (i, 0)),
            pl.BlockSpec((D_IN, H), lambda i: (0, 0)),
        ],
        out_specs=pl.BlockSpec((BLK, H), lambda i: (i, 0)),
        out_shape=jax.ShapeDtypeStruct((NPAD, H), jnp.float32),
    )(x_p, W1)


def _tc_dinv_t1(degacc, u1):
    def body(deg_ref, u_ref, dinv_ref, t_ref):
        i = pl.program_id(0)
        d = deg_ref[0, :, 0:1] + deg_ref[1, :, 0:1] + 1.0
        rows = lax.broadcasted_iota(jnp.int32, (BLK, 1), 0) + i * BLK
        dinv = jnp.where(rows < N, lax.rsqrt(d), 0.0)
        dinv_ref[...] = dinv
        t_ref[...] = u_ref[...] * dinv

    return pl.pallas_call(
        body,
        grid=(GRID,),
        in_specs=[
            pl.BlockSpec((2, BLK, DEGW), lambda i: (0, i, 0)),
            pl.BlockSpec((BLK, H), lambda i: (i, 0)),
        ],
        out_specs=[
            pl.BlockSpec((BLK, 1), lambda i: (i, 0)),
            pl.BlockSpec((BLK, H), lambda i: (i, 0)),
        ],
        out_shape=[
            jax.ShapeDtypeStruct((NPAD, 1), jnp.float32),
            jax.ShapeDtypeStruct((NPAD, H), jnp.float32),
        ],
    )(degacc, u1)


def _tc_mid(acc, t, dinv, b, Wn):
    def body(a_ref, t_ref, dinv_ref, b_ref, w_ref, o_ref):
        z = (a_ref[0] + a_ref[1] + t_ref[...]) * dinv_ref[...] + b_ref[...]
        z = jnp.maximum(z, 0.0)
        u = jnp.dot(z, w_ref[...], preferred_element_type=jnp.float32)
        o_ref[...] = u * dinv_ref[...]

    return pl.pallas_call(
        body,
        grid=(GRID,),
        in_specs=[
            pl.BlockSpec((2, BLK, H), lambda i: (0, i, 0)),
            pl.BlockSpec((BLK, H), lambda i: (i, 0)),
            pl.BlockSpec((BLK, 1), lambda i: (i, 0)),
            pl.BlockSpec((1, H), lambda i: (0, 0)),
            pl.BlockSpec((H, H), lambda i: (0, 0)),
        ],
        out_specs=pl.BlockSpec((BLK, H), lambda i: (i, 0)),
        out_shape=jax.ShapeDtypeStruct((NPAD, H), jnp.float32),
    )(acc, t, dinv, b, Wn)


def _tc_final(acc, t, dinv, b3, batch2d, Wo, bo):
    def body(a_ref, t_ref, dinv_ref, b_ref, batch_ref, wo_ref, bo_ref,
             o_ref, pooled):
        i = pl.program_id(0)
        z = (a_ref[0] + a_ref[1] + t_ref[...]) * dinv_ref[...] + b_ref[...]
        z = jnp.maximum(z, 0.0)
        rows = lax.broadcasted_iota(jnp.int32, (BLK, 1), 0) + i * BLK
        z = jnp.where(rows < N, z, 0.0)
        segs = lax.broadcasted_iota(jnp.int32, (G, 1), 0)
        oh = (segs == batch_ref[...]).astype(jnp.float32)
        contrib = jnp.dot(oh, z, preferred_element_type=jnp.float32)

        @pl.when(i == 0)
        def _():
            pooled[...] = contrib

        @pl.when(i > 0)
        def _():
            pooled[...] = pooled[...] + contrib

        @pl.when(i == GRID - 1)
        def _():
            o_ref[...] = jnp.dot(pooled[...], wo_ref[...],
                                 preferred_element_type=jnp.float32) + bo_ref[...]

    return pl.pallas_call(
        body,
        grid=(GRID,),
        in_specs=[
            pl.BlockSpec((2, BLK, H), lambda i: (0, i, 0)),
            pl.BlockSpec((BLK, H), lambda i: (i, 0)),
            pl.BlockSpec((BLK, 1), lambda i: (i, 0)),
            pl.BlockSpec((1, H), lambda i: (0, 0)),
            pl.BlockSpec((1, BLK), lambda i: (0, i)),
            pl.BlockSpec((H, 1), lambda i: (0, 0)),
            pl.BlockSpec((1, 1), lambda i: (0, 0)),
        ],
        out_specs=pl.BlockSpec((G, 1), lambda i: (0, 0)),
        out_shape=jax.ShapeDtypeStruct((G, 1), jnp.float32),
        scratch_shapes=[pltpu.VMEM((G, H), jnp.float32)],
    )(acc, t, dinv, b3, batch2d, Wo, bo)


def kernel(x, edge_index, batch, W1, b1, W2, b2, W3, b3, Wo, bo):
    x_p = jnp.zeros((NPAD, D_IN), jnp.float32).at[:N].set(x)

    src = edge_index[0]
    dst = edge_index[1]
    def _part(e):
        e2 = jnp.full((NT * NCHUNK * CH,), N, jnp.int32).at[:E].set(e)
        e2 = e2.reshape(NT, NCHUNK * CH)
        pad = jnp.full((NT, (NALLOC - NCHUNK) * CH), N, jnp.int32)
        return jnp.concatenate([e2, pad], axis=1).reshape(NT, NALLOC, CH)

    src3d = _part(src)
    dst3d = _part(dst)

    batch2d = jnp.zeros((1, NPAD), jnp.int32).at[0, :N].set(batch)

    b1r = b1.reshape(1, H)
    b2r = b2.reshape(1, H)
    b3r = b3.reshape(1, H)
    bor = bo.reshape(1, 1)

    u1 = _tc_u1(x_p, W1)
    degacc = _sc_deg(dst3d)
    dinv, t1 = _tc_dinv_t1(degacc, u1)
    a1 = _sc_prop(t1, src3d, dst3d)
    t2 = _tc_mid(a1, t1, dinv, b1r, W2)
    a2 = _sc_prop(t2, src3d, dst3d)
    t3 = _tc_mid(a2, t2, dinv, b2r, W3)
    a3 = _sc_prop(t3, src3d, dst3d)
    return _tc_final(a3, t3, dinv, b3r, batch2d, Wo, bor)


# speedup vs baseline: 2.8051x; 1.0307x over previous
import functools

import jax
import jax.numpy as jnp
from jax import lax
from jax.experimental import pallas as pl
from jax.experimental.pallas import tpu as pltpu
from jax.experimental.pallas import tpu_sc as plsc

N = 10000
E = 640000
D_IN = 128
H = 64
G = 64

NPAD = 10240
BLK = 512
GRID = NPAD // BLK

NT = 32
CH = 128
NB = 5
IB = 10
NCHUNK = 160
NBLK = NCHUNK // IB
NALLOC = NCHUNK + IB
RPT = NPAD // 16
CBP = NBLK // 2
DEGW = 16

_mesh = plsc.VectorSubcoreMesh(core_axis_name="c", subcore_axis_name="s")


def _sc_deg(dst3d):
    @functools.partial(
        pl.kernel,
        mesh=_mesh,
        out_type=jax.ShapeDtypeStruct((2, NPAD, DEGW), jnp.float32),
        compiler_params=pltpu.CompilerParams(use_tc_tiling_on_sc=False),
        scratch_types=[
            pltpu.VMEM((NALLOC, CH), jnp.int32),
            pltpu.VMEM((CH, DEGW), jnp.float32),
            pltpu.VMEM((64, DEGW), jnp.float32),
            pltpu.VMEM_SHARED((NPAD, DEGW), jnp.float32),
            pltpu.SemaphoreType.DMA,
            pltpu.SemaphoreType.DMA,
        ],
    )
    def k(dst_ref, out_ref, dst_all, ones, zb, acc, isem, ssem):
        c = lax.axis_index("c")
        s = lax.axis_index("s")
        blk = c * 16 + s

        idx_cp = pltpu.async_copy(dst_ref.at[blk], dst_all, isem)

        @pl.loop(0, 64)
        def _(r):
            zb.at[r][pl.ds(0, 16)] = jnp.zeros((16,), jnp.float32)

        @pl.loop(0, CH)
        def _(r):
            ones.at[r][pl.ds(0, 16)] = jnp.ones((16,), jnp.float32)

        @pl.loop(0, RPT, step=64)
        def _(r0):
            pltpu.sync_copy(zb, acc.at[pl.ds(s * RPT + r0, 64)])

        idx_cp.wait()
        plsc.subcore_barrier()

        @pl.loop(0, NALLOC // 8)
        def _(g):
            base = g * 8
            for kk in range(8):
                pltpu.async_copy(ones, acc.at[dst_all.at[base + kk]], ssem,
                                 add=True)
            for kk in range(8):
                pltpu.make_async_copy(ones, acc.at[dst_all.at[base + kk]],
                                      ssem).wait()

        plsc.subcore_barrier()
        pltpu.sync_copy(acc.at[pl.ds(s * RPT, RPT)],
                        out_ref.at[c, pl.ds(s * RPT, RPT)])

    return k(dst3d)


def _sc_prop(t_hbm, src3d, dst3d):
    @functools.partial(
        pl.kernel,
        mesh=_mesh,
        out_type=jax.ShapeDtypeStruct((2, NPAD, H), jnp.float32),
        compiler_params=pltpu.CompilerParams(use_tc_tiling_on_sc=False),
        scratch_types=(
            [pltpu.VMEM((IB, CH), jnp.int32)] * 4
            + [pltpu.VMEM((CH, H), jnp.float32)] * NB
            + [pltpu.VMEM((32, H), jnp.float32)]
            + [pltpu.VMEM_SHARED((NPAD, H), jnp.float32)]
            + [pltpu.VMEM_SHARED((NPAD, H), jnp.float32)]
            + [pltpu.SemaphoreType.DMA] * (2 * NB + 2)
        ),
    )
    def k(t_ref, src_ref, dst_ref, out_ref, *sc):
        sidx = sc[0:2]
        didx = sc[2:4]
        rows = sc[4:4 + NB]
        zb = sc[4 + NB]
        acc = sc[5 + NB]
        t_sp = sc[6 + NB]
        gsems = sc[7 + NB:7 + 2 * NB]
        ssems = sc[7 + 2 * NB:7 + 3 * NB]
        isem = sc[7 + 3 * NB]
        stsem = sc[8 + 3 * NB]

        c = lax.axis_index("c")
        s = lax.axis_index("s")
        blk = c * 16 + s

        stage_cp = pltpu.async_copy(t_ref.at[pl.ds(s * RPT, RPT)],
                                    t_sp.at[pl.ds(s * RPT, RPT)], stsem)
        s0_cp = pltpu.async_copy(src_ref.at[blk, pl.ds(0, IB)], sidx[0], isem)
        d0_cp = pltpu.async_copy(dst_ref.at[blk, pl.ds(0, IB)], didx[0], isem)

        @pl.loop(0, 32)
        def _(r):
            @pl.loop(0, H, step=16)
            def _(k2):
                zb.at[r][pl.ds(k2, 16)] = jnp.zeros((16,), jnp.float32)

        for r0 in range(0, RPT, 32):
            pltpu.async_copy(zb, acc.at[pl.ds(s * RPT + r0, 32)], ssems[0])
        for r0 in range(0, RPT, 32):
            pltpu.make_async_copy(zb, acc.at[pl.ds(s * RPT + r0, 32)],
                                  ssems[0]).wait()

        stage_cp.wait()
        s0_cp.wait()
        d0_cp.wait()
        plsc.subcore_barrier()

        for b in range(NB):
            pltpu.async_copy(t_sp.at[sidx[0].at[b]], rows[b], gsems[b])

        def nb_group(gref, cur_s, cur_d, nxt_s, nxt_d, j0, j0n):
            for b in range(NB):
                pltpu.make_async_copy(gref.at[cur_s.at[j0 + b]], rows[b],
                                      gsems[b]).wait()
                pltpu.async_copy(rows[b], acc.at[cur_d.at[j0 + b]], ssems[b],
                                 add=True)
            for b in range(NB):
                pltpu.make_async_copy(rows[b], acc.at[cur_d.at[j0 + b]],
                                      ssems[b]).wait()
                pltpu.async_copy(gref.at[nxt_s.at[j0n + b]], rows[b],
                                 gsems[b])

        def block_pair(kk, gref):
            for par in range(2):
                k2 = kk * 2 + par
                cur_s, cur_d = sidx[par], didx[par]
                oth_s, oth_d = sidx[1 - par], didx[1 - par]
                base = k2 * IB
                pltpu.async_copy(src_ref.at[blk, pl.ds(base + IB, IB)],
                                 oth_s, isem)
                pltpu.async_copy(dst_ref.at[blk, pl.ds(base + IB, IB)],
                                 oth_d, isem)
                for j0 in range(0, IB - NB, NB):
                    nb_group(gref, cur_s, cur_d, cur_s, cur_d, j0, j0 + NB)
                pltpu.make_async_copy(src_ref.at[blk, pl.ds(base + IB, IB)],
                                      oth_s, isem).wait()
                pltpu.make_async_copy(dst_ref.at[blk, pl.ds(base + IB, IB)],
                                      oth_d, isem).wait()
                nb_group(gref, cur_s, cur_d, oth_s, oth_d, IB - NB, 0)

        @pl.loop(0, CBP)
        def _(kk):
            block_pair(kk, t_sp)

        @pl.loop(CBP, NBLK // 2)
        def _(kk):
            block_pair(kk, t_ref)

        for b in range(NB):
            pltpu.make_async_copy(t_sp.at[sidx[0].at[b]], rows[b],
                                  gsems[b]).wait()

        plsc.subcore_barrier()
        pltpu.sync_copy(acc.at[pl.ds(s * RPT, RPT)],
                        out_ref.at[c, pl.ds(s * RPT, RPT)])

    return k(t_hbm, src3d, dst3d)


def _tc_u1(x_p, W1):
    def body(x_ref, w_ref, u_ref):
        u_ref[...] = jnp.dot(x_ref[...], w_ref[...],
                             preferred_element_type=jnp.float32)

    return pl.pallas_call(
        body,
        grid=(GRID,),
        in_specs=[
            pl.BlockSpec((BLK, D_IN), lambda i: (i, 0)),
            pl.BlockSpec((D_IN, H), lambda i: (0, 0)),
        ],
        out_specs=pl.BlockSpec((BLK, H), lambda i: (i, 0)),
        out_shape=jax.ShapeDtypeStruct((NPAD, H), jnp.float32),
    )(x_p, W1)


def _tc_dinv_t1(degacc, u1):
    def body(deg_ref, u_ref, dinv_ref, t_ref):
        i = pl.program_id(0)
        d = deg_ref[0, :, 0:1] + deg_ref[1, :, 0:1] + 1.0
        rows = lax.broadcasted_iota(jnp.int32, (BLK, 1), 0) + i * BLK
        dinv = jnp.where(rows < N, lax.rsqrt(d), 0.0)
        dinv_ref[...] = dinv
        t_ref[...] = u_ref[...] * dinv

    return pl.pallas_call(
        body,
        grid=(GRID,),
        in_specs=[
            pl.BlockSpec((2, BLK, DEGW), lambda i: (0, i, 0)),
            pl.BlockSpec((BLK, H), lambda i: (i, 0)),
        ],
        out_specs=[
            pl.BlockSpec((BLK, 1), lambda i: (i, 0)),
            pl.BlockSpec((BLK, H), lambda i: (i, 0)),
        ],
        out_shape=[
            jax.ShapeDtypeStruct((NPAD, 1), jnp.float32),
            jax.ShapeDtypeStruct((NPAD, H), jnp.float32),
        ],
    )(degacc, u1)


def _tc_mid(acc, t, dinv, b, Wn):
    def body(a_ref, t_ref, dinv_ref, b_ref, w_ref, o_ref):
        z = (a_ref[0] + a_ref[1] + t_ref[...]) * dinv_ref[...] + b_ref[...]
        z = jnp.maximum(z, 0.0)
        u = jnp.dot(z, w_ref[...], preferred_element_type=jnp.float32)
        o_ref[...] = u * dinv_ref[...]

    return pl.pallas_call(
        body,
        grid=(GRID,),
        in_specs=[
            pl.BlockSpec((2, BLK, H), lambda i: (0, i, 0)),
            pl.BlockSpec((BLK, H), lambda i: (i, 0)),
            pl.BlockSpec((BLK, 1), lambda i: (i, 0)),
            pl.BlockSpec((1, H), lambda i: (0, 0)),
            pl.BlockSpec((H, H), lambda i: (0, 0)),
        ],
        out_specs=pl.BlockSpec((BLK, H), lambda i: (i, 0)),
        out_shape=jax.ShapeDtypeStruct((NPAD, H), jnp.float32),
    )(acc, t, dinv, b, Wn)


def _tc_final(acc, t, dinv, b3, batch2d, Wo, bo):
    def body(a_ref, t_ref, dinv_ref, b_ref, batch_ref, wo_ref, bo_ref,
             o_ref, pooled):
        i = pl.program_id(0)
        z = (a_ref[0] + a_ref[1] + t_ref[...]) * dinv_ref[...] + b_ref[...]
        z = jnp.maximum(z, 0.0)
        rows = lax.broadcasted_iota(jnp.int32, (BLK, 1), 0) + i * BLK
        z = jnp.where(rows < N, z, 0.0)
        segs = lax.broadcasted_iota(jnp.int32, (G, 1), 0)
        oh = (segs == batch_ref[...]).astype(jnp.float32)
        contrib = jnp.dot(oh, z, preferred_element_type=jnp.float32)

        @pl.when(i == 0)
        def _():
            pooled[...] = contrib

        @pl.when(i > 0)
        def _():
            pooled[...] = pooled[...] + contrib

        @pl.when(i == GRID - 1)
        def _():
            o_ref[...] = jnp.dot(pooled[...], wo_ref[...],
                                 preferred_element_type=jnp.float32) + bo_ref[...]

    return pl.pallas_call(
        body,
        grid=(GRID,),
        in_specs=[
            pl.BlockSpec((2, BLK, H), lambda i: (0, i, 0)),
            pl.BlockSpec((BLK, H), lambda i: (i, 0)),
            pl.BlockSpec((BLK, 1), lambda i: (i, 0)),
            pl.BlockSpec((1, H), lambda i: (0, 0)),
            pl.BlockSpec((1, BLK), lambda i: (0, i)),
            pl.BlockSpec((H, 1), lambda i: (0, 0)),
            pl.BlockSpec((1, 1), lambda i: (0, 0)),
        ],
        out_specs=pl.BlockSpec((G, 1), lambda i: (0, 0)),
        out_shape=jax.ShapeDtypeStruct((G, 1), jnp.float32),
        scratch_shapes=[pltpu.VMEM((G, H), jnp.float32)],
    )(acc, t, dinv, b3, batch2d, Wo, bo)


def kernel(x, edge_index, batch, W1, b1, W2, b2, W3, b3, Wo, bo):
    x_p = jnp.zeros((NPAD, D_IN), jnp.float32).at[:N].set(x)

    src = edge_index[0]
    dst = edge_index[1]
    def _part(e):
        e2 = jnp.full((NT * NCHUNK * CH,), N, jnp.int32).at[:E].set(e)
        e2 = e2.reshape(NT, NCHUNK * CH)
        pad = jnp.full((NT, (NALLOC - NCHUNK) * CH), N, jnp.int32)
        return jnp.concatenate([e2, pad], axis=1).reshape(NT, NALLOC, CH)

    src3d = _part(src)
    dst3d = _part(dst)

    batch2d = jnp.zeros((1, NPAD), jnp.int32).at[0, :N].set(batch)

    b1r = b1.reshape(1, H)
    b2r = b2.reshape(1, H)
    b3r = b3.reshape(1, H)
    bor = bo.reshape(1, 1)

    u1 = _tc_u1(x_p, W1)
    degacc = _sc_deg(dst3d)
    dinv, t1 = _tc_dinv_t1(degacc, u1)
    a1 = _sc_prop(t1, src3d, dst3d)
    t2 = _tc_mid(a1, t1, dinv, b1r, W2)
    a2 = _sc_prop(t2, src3d, dst3d)
    t3 = _tc_mid(a2, t2, dinv, b2r, W3)
    a3 = _sc_prop(t3, src3d, dst3d)
    return _tc_final(a3, t3, dinv, b3r, batch2d, Wo, bor)
